# baseline (device time: 16638 ns/iter reference)
import jax
import jax.numpy as jnp
from jax import lax
from jax.experimental import pallas as pl
from jax.experimental.pallas import tpu as pltpu

EPS = 1e-5
GLOBAL_N = 2048
NCHUNK = 8
LAG = 2


def kernel(x, gamma):
    m, n = x.shape
    gamma2d = gamma.reshape(1, n)
    cm = m // NCHUNK

    def body(x_blk, g_ref, o_blk, x_keep, send_buf, recv_buf,
             send_sems, recv_sems):
        c = pl.program_id(0)
        my_x = lax.axis_index("x")
        my_y = lax.axis_index("y")
        peer = (my_x, 1 - my_y)

        @pl.when(c == 0)
        def _():
            barrier_sem = pltpu.get_barrier_semaphore()
            pl.semaphore_signal(
                barrier_sem, inc=1, device_id=peer,
                device_id_type=pl.DeviceIdType.MESH,
            )
            pl.semaphore_wait(barrier_sem, 1)

        @pl.when(c < NCHUNK)
        def _():
            seg = pl.ds(c * cm, cm)
            xc = x_blk[:, :]
            x_keep[seg, :] = xc
            ones = jnp.ones((1, n), dtype=jnp.float32)
            s = lax.dot_general(
                ones, xc * xc,
                dimension_numbers=(((1,), (1,)), ((), ())),
                preferred_element_type=jnp.float32,
            )
            send_buf[:, seg] = s
            pltpu.make_async_remote_copy(
                src_ref=send_buf.at[:, seg],
                dst_ref=recv_buf.at[:, seg],
                send_sem=send_sems.at[c],
                recv_sem=recv_sems.at[c],
                device_id=peer,
                device_id_type=pl.DeviceIdType.MESH,
            ).start()

        @pl.when(c >= LAG)
        def _():
            d = c - LAG
            seg = pl.ds(d * cm, cm)
            rdma = pltpu.make_async_remote_copy(
                src_ref=send_buf.at[:, seg],
                dst_ref=recv_buf.at[:, seg],
                send_sem=send_sems.at[d],
                recv_sem=recv_sems.at[d],
                device_id=peer,
                device_id_type=pl.DeviceIdType.MESH,
            )
            rdma.wait_send()
            rdma.wait_recv()
            total = send_buf[:, seg] + recv_buf[:, seg]
            inv_rms = lax.rsqrt(total * (1.0 / GLOBAL_N) + EPS)
            o_blk[:, :] = x_keep[seg, :] * inv_rms.reshape(cm, 1) * g_ref[:, :]

    grid = (NCHUNK + LAG,)
    return pl.pallas_call(
        body,
        grid=grid,
        out_shape=jax.ShapeDtypeStruct((m, n), x.dtype),
        in_specs=[
            pl.BlockSpec(
                (cm, n), lambda c: (jnp.minimum(c, NCHUNK - 1), 0)
            ),
            pl.BlockSpec((1, n), lambda c: (0, 0)),
        ],
        out_specs=pl.BlockSpec(
            (cm, n), lambda c: (jnp.clip(c - LAG, 0, NCHUNK - 1), 0)
        ),
        scratch_shapes=[
            pltpu.VMEM((m, n), jnp.float32),
            pltpu.VMEM((1, m), jnp.float32),
            pltpu.VMEM((1, m), jnp.float32),
            pltpu.SemaphoreType.DMA((NCHUNK,)),
            pltpu.SemaphoreType.DMA((NCHUNK,)),
        ],
        compiler_params=pltpu.CompilerParams(collective_id=0),
    )(x, gamma2d)


# device time: 12993 ns/iter; 1.2805x vs baseline; 1.2805x over previous
import jax
import jax.numpy as jnp
from jax import lax
from jax.experimental import pallas as pl
from jax.experimental.pallas import tpu as pltpu

EPS = 1e-5
GLOBAL_N = 2048
NCHUNK = 4


def kernel(x, gamma):
    m, n = x.shape
    gamma2d = gamma.reshape(1, n)
    cm = m // NCHUNK

    def body(x_ref, g_ref, o_ref, send_buf, recv_buf, send_sems, recv_sems):
        my_x = lax.axis_index("x")
        my_y = lax.axis_index("y")
        peer = (my_x, 1 - my_y)

        barrier_sem = pltpu.get_barrier_semaphore()
        pl.semaphore_signal(
            barrier_sem, inc=1, device_id=peer,
            device_id_type=pl.DeviceIdType.MESH,
        )
        pl.semaphore_wait(barrier_sem, 1)

        ones = jnp.ones((1, n), dtype=jnp.float32)
        rdmas = []
        for c in range(NCHUNK):
            seg = pl.ds(c * cm, cm)
            xc = x_ref[seg, :]
            s = lax.dot_general(
                ones, xc * xc,
                dimension_numbers=(((1,), (1,)), ((), ())),
                preferred_element_type=jnp.float32,
            )
            send_buf[:, seg] = s
            rdma = pltpu.make_async_remote_copy(
                src_ref=send_buf.at[:, seg],
                dst_ref=recv_buf.at[:, seg],
                send_sem=send_sems.at[c],
                recv_sem=recv_sems.at[c],
                device_id=peer,
                device_id_type=pl.DeviceIdType.MESH,
            )
            rdma.start()
            rdmas.append(rdma)

        for c in range(NCHUNK):
            rdmas[c].wait_recv()
            seg = pl.ds(c * cm, cm)
            total = send_buf[:, seg] + recv_buf[:, seg]
            inv_rms = lax.rsqrt(total * (1.0 / GLOBAL_N) + EPS)
            o_ref[seg, :] = x_ref[seg, :] * inv_rms.reshape(cm, 1) * g_ref[:, :]

        for c in range(NCHUNK):
            rdmas[c].wait_send()

    return pl.pallas_call(
        body,
        out_shape=jax.ShapeDtypeStruct((m, n), x.dtype),
        in_specs=[
            pl.BlockSpec(memory_space=pltpu.VMEM),
            pl.BlockSpec(memory_space=pltpu.VMEM),
        ],
        out_specs=pl.BlockSpec(memory_space=pltpu.VMEM),
        scratch_shapes=[
            pltpu.VMEM((1, m), jnp.float32),
            pltpu.VMEM((1, m), jnp.float32),
            pltpu.SemaphoreType.DMA((NCHUNK,)),
            pltpu.SemaphoreType.DMA((NCHUNK,)),
        ],
        compiler_params=pltpu.CompilerParams(collective_id=0),
    )(x, gamma2d)
